# baseline (speedup 1.0000x reference)
import functools

import jax
import jax.numpy as jnp
from jax import lax
from jax.experimental import pallas as pl
from jax.experimental.pallas import tpu as pltpu
from jax.experimental.pallas import tpu_sc as plsc

N = 10000
D = 128
NPAD = 10112
DUMMY = 10016
ROWS_PT = NPAD // 16

E = 320000
CH = 96
NCH0 = 128
NCH1 = 84
EPAD = 16 * (NCH0 + NCH1) * CH

ED = 200000
CHD = 64
NCHD = 98
EDPT = NCHD * CHD
EDPAD = 32 * EDPT

_mesh = plsc.VectorSubcoreMesh(core_axis_name="c", subcore_axis_name="s")

_GDN = lax.GatherDimensionNumbers(
    offset_dims=(), collapsed_slice_dims=(0,), start_index_map=(0,))


def _permute(v, idx):
    return lax.gather(v, idx[:, None], _GDN, (1,),
                      mode=lax.GatherScatterMode.PROMISE_IN_BOUNDS)


def _agg_body(src_hbm, dst_hbm, x_hbm, zeros_hbm,
              agg_out,
              sidx_a, sidx_b, didx_a, didx_b, rows_a, rows_b, agg_sh,
              gsem_a, gsem_b, ssem_a, ssem_b, dsem_a, dsem_b):
    c = lax.axis_index("c")
    s = lax.axis_index("s")
    w = c * 16 + s
    rbase = s * ROWS_PT
    chunks = ((0, 96), (96, 96), (192, 96), (288, 96), (384, 96),
              (480, 96), (576, 56))
    for off, n in chunks:
        pltpu.sync_copy(zeros_hbm.at[pl.ds(rbase + off, n)],
                        rows_a.at[pl.ds(0, n)])
        pltpu.sync_copy(rows_a.at[pl.ds(0, n)],
                        agg_sh.at[pl.ds(rbase + off, n)])
    plsc.subcore_barrier()

    ebase = (s * (NCH0 + NCH1) + c * NCH0) * CH
    m = jnp.where(c == 0, NCH0, NCH1)
    sbufs = (sidx_a, sidx_b)
    dbufs = (didx_a, didx_b)
    rbufs = (rows_a, rows_b)
    gsems = (gsem_a, gsem_b)
    ssems = (ssem_a, ssem_b)
    dsems = (dsem_a, dsem_b)

    def sslice(i):
        return src_hbm.at[pl.ds(pl.multiple_of(ebase + i * CH, 8), CH)]

    def dslice(i):
        return dst_hbm.at[pl.ds(pl.multiple_of(ebase + i * CH, 8), CH)]

    def step(i, p):
        q = 1 - p
        pltpu.make_async_copy(sslice(i + 1), sbufs[q], ssems[q]).wait()
        pltpu.async_copy(x_hbm.at[sbufs[q]], rbufs[q], gsems[q])
        pltpu.make_async_copy(x_hbm.at[sbufs[p]], rbufs[p], gsems[p]).wait()
        pltpu.make_async_copy(dslice(i), dbufs[p], dsems[p]).wait()
        pltpu.sync_copy(rbufs[p], agg_sh.at[dbufs[p]], add=True)
        pltpu.async_copy(sslice(i + 2), sbufs[p], ssems[p])
        pltpu.async_copy(dslice(i + 2), dbufs[p], dsems[p])

    pltpu.sync_copy(sslice(0), sidx_a)
    pltpu.async_copy(dslice(0), didx_a, dsem_a)
    pltpu.async_copy(x_hbm.at[sidx_a], rows_a, gsem_a)
    pltpu.async_copy(sslice(1), sidx_b, ssem_b)
    pltpu.async_copy(dslice(1), didx_b, dsem_b)

    def body(g, carry):
        i = g * 2
        step(i, 0)
        step(i + 1, 1)
        return carry

    lax.fori_loop(0, m // 2 - 1, body, 0)
    step(m - 2, 0)
    pltpu.make_async_copy(x_hbm.at[sidx_b], rows_b, gsem_b).wait()
    pltpu.make_async_copy(dslice(m - 1), didx_b, dsem_b).wait()
    pltpu.sync_copy(rows_b, agg_sh.at[didx_b], add=True)
    pltpu.make_async_copy(sslice(m), sidx_a, ssem_a).wait()
    pltpu.make_async_copy(dslice(m), didx_a, dsem_a).wait()

    plsc.subcore_barrier()
    for off, n in chunks:
        pltpu.sync_copy(agg_sh.at[pl.ds(rbase + off, n)],
                        rows_a.at[pl.ds(0, n)])
        pltpu.sync_copy(rows_a.at[pl.ds(0, n)],
                        agg_out.at[c, pl.ds(rbase + off, n)])


_agg_call = pl.kernel(
    _agg_body,
    out_type=jax.ShapeDtypeStruct((2, NPAD, D), jnp.float32),
    mesh=_mesh,
    scratch_types=[
        pltpu.VMEM((CH,), jnp.int32),
        pltpu.VMEM((CH,), jnp.int32),
        pltpu.VMEM((CH,), jnp.int32),
        pltpu.VMEM((CH,), jnp.int32),
        pltpu.VMEM((CH, D), jnp.float32),
        pltpu.VMEM((CH, D), jnp.float32),
        pltpu.VMEM_SHARED((NPAD, D), jnp.float32),
        pltpu.SemaphoreType.DMA,
        pltpu.SemaphoreType.DMA,
        pltpu.SemaphoreType.DMA,
        pltpu.SemaphoreType.DMA,
        pltpu.SemaphoreType.DMA,
        pltpu.SemaphoreType.DMA,
    ],
)


def _decode_body(esd_hbm, pq_hbm, w2_hbm, b2_hbm,
                 out_hbm,
                 idx_a, idx_b, rows_a, rows_b, w2v, b2v, sc_v,
                 gsem_a, gsem_b, isem_a, isem_b):
    c = lax.axis_index("c")
    s = lax.axis_index("s")
    w = c * 16 + s
    ebase = w * EDPT
    pltpu.sync_copy(w2_hbm, w2v)
    pltpu.sync_copy(b2_hbm, b2v)
    w2r = [w2v[pl.ds(16 * k, 16)] for k in range(8)]
    b2r = b2v[...]
    lanes = lax.broadcasted_iota(jnp.int32, (16,), 0)
    ibufs = (idx_a, idx_b)
    isems = (isem_a, isem_b)
    rbufs = (rows_a, rows_b)
    gsems = (gsem_a, gsem_b)

    def isl(i):
        return esd_hbm.at[pl.ds(
            pl.multiple_of(2 * (ebase + i * CHD), 8), 2 * CHD)]

    def step(i, p):
        q = 1 - p
        pltpu.make_async_copy(isl(i + 1), ibufs[q], isems[q]).wait()
        pltpu.async_copy(pq_hbm.at[ibufs[q]], rbufs[q], gsems[q])
        pltpu.make_async_copy(pq_hbm.at[ibufs[p]], rbufs[p], gsems[p]).wait()
        pltpu.async_copy(isl(i + 2), ibufs[p], isems[p])
        rows = rbufs[p]

        def group(g, carry2):
            vec = jnp.zeros((16,), jnp.float32)
            for j in range(16):
                e = g * 16 + j
                acc = None
                for k in range(8):
                    pv = rows[2 * e, pl.ds(16 * k, 16)]
                    qv = rows[2 * e + 1, pl.ds(16 * k, 16)]
                    t = jnp.maximum(pv + qv, 0.0) * w2r[k]
                    acc = t if acc is None else acc + t
                for sh in (8, 4, 2, 1):
                    acc = acc + _permute(acc, lanes ^ sh)
                vec = jnp.where(lanes == j, acc, vec)
            vec = vec + b2r
            vec = 1.0 / (1.0 + jnp.exp(-vec))
            sc_v[pl.ds(g * 16, 16)] = vec
            return carry2

        lax.fori_loop(0, CHD // 16, group, 0)
        pltpu.sync_copy(sc_v, out_hbm.at[pl.ds(
            pl.multiple_of(ebase + i * CHD, 8), CHD)])

    pltpu.sync_copy(isl(0), idx_a)
    pltpu.async_copy(pq_hbm.at[idx_a], rows_a, gsem_a)
    pltpu.async_copy(isl(1), idx_b, isem_b)

    def chunk(g, carry):
        i = g * 2
        step(i, 0)
        step(i + 1, 1)
        return carry

    lax.fori_loop(0, NCHD // 2, chunk, 0)
    pltpu.make_async_copy(pq_hbm.at[idx_a], rows_a, gsem_a).wait()
    pltpu.make_async_copy(isl(NCHD + 1), idx_b, isem_b).wait()


_decode_call = pl.kernel(
    _decode_body,
    out_type=jax.ShapeDtypeStruct((EDPAD,), jnp.float32),
    mesh=_mesh,
    scratch_types=[
        pltpu.VMEM((2 * CHD,), jnp.int32),
        pltpu.VMEM((2 * CHD,), jnp.int32),
        pltpu.VMEM((2 * CHD, D), jnp.float32),
        pltpu.VMEM((2 * CHD, D), jnp.float32),
        pltpu.VMEM((D,), jnp.float32),
        pltpu.VMEM((16,), jnp.float32),
        pltpu.VMEM((CHD,), jnp.float32),
        pltpu.SemaphoreType.DMA,
        pltpu.SemaphoreType.DMA,
        pltpu.SemaphoreType.DMA,
        pltpu.SemaphoreType.DMA,
    ],
)


def _mean_from_parts(agg_ref, cnt_ref):
    a = agg_ref[0, :N, :] + agg_ref[1, :N, :]
    cnt = cnt_ref[0, :N, 0:1] + cnt_ref[1, :N, 0:1]
    return a / jnp.maximum(cnt, 1.0)


def _tc_layer_body(agg_ref, cnt_ref, h_ref, wl_ref, bl_ref, wr_ref,
                   g_ref, be_ref, out_ref):
    mean = _mean_from_parts(agg_ref, cnt_ref)
    h = h_ref[...]
    y = (jnp.dot(mean, wl_ref[...], preferred_element_type=jnp.float32)
         + bl_ref[...]
         + jnp.dot(h, wr_ref[...], preferred_element_type=jnp.float32))
    mu = jnp.mean(y, axis=0, keepdims=True)
    dlt = y - mu
    var = jnp.mean(dlt * dlt, axis=0, keepdims=True)
    yn = dlt * lax.rsqrt(var + 1e-5) * g_ref[...] + be_ref[...]
    out_ref[...] = jnp.maximum(yn, 0.0) + h


_tc_layer_call = pl.pallas_call(
    _tc_layer_body,
    out_shape=jax.ShapeDtypeStruct((N, D), jnp.float32),
)


def _tc_final_body(agg_ref, cnt_ref, h_ref, wl_ref, bl_ref, wr_ref,
                   w1a_ref, w1b_ref, b1_ref,
                   z_ref, pq_ref):
    mean = _mean_from_parts(agg_ref, cnt_ref)
    z = (jnp.dot(mean, wl_ref[...], preferred_element_type=jnp.float32)
         + bl_ref[...]
         + jnp.dot(h_ref[...], wr_ref[...], preferred_element_type=jnp.float32))
    z_ref[...] = z
    pq_ref[0] = jnp.dot(z, w1a_ref[...], preferred_element_type=jnp.float32)
    pq_ref[1] = (jnp.dot(z, w1b_ref[...], preferred_element_type=jnp.float32)
                 + b1_ref[...])


_tc_final_call = pl.pallas_call(
    _tc_final_body,
    out_shape=(jax.ShapeDtypeStruct((N, D), jnp.float32),
               jax.ShapeDtypeStruct((2, N, D), jnp.float32)),
)


def kernel(x, edge_index, pos_edge_index, neg_edge_index,
           Wl0, bl0, Wr0, Wl1, bl1, Wr1, Wl2, bl2, Wr2,
           g0, be0, g1, be1, ep_W1, ep_b1, ep_W2, ep_b2):
    i32 = jnp.int32
    src = edge_index[0].astype(i32)
    dst = edge_index[1].astype(i32)
    src_p = jnp.concatenate([src, jnp.zeros((EPAD + CH - E,), i32)])
    dst_p = jnp.concatenate([dst, jnp.full((EPAD - E,), DUMMY, i32),
                             jnp.zeros((CH,), i32)])
    zeros = jnp.zeros((NPAD, D), jnp.float32)
    ones_mat = jnp.ones((N, D), jnp.float32)

    bl0r, bl1r, bl2r = bl0.reshape(1, D), bl1.reshape(1, D), bl2.reshape(1, D)
    g0r, be0r = g0.reshape(1, D), be0.reshape(1, D)
    g1r, be1r = g1.reshape(1, D), be1.reshape(1, D)

    cntp = _agg_call(src_p, dst_p, ones_mat, zeros)
    agg0 = _agg_call(src_p, dst_p, x, zeros)
    h0 = _tc_layer_call(agg0, cntp, x, Wl0, bl0r, Wr0, g0r, be0r)
    agg1 = _agg_call(src_p, dst_p, h0, zeros)
    h1 = _tc_layer_call(agg1, cntp, h0, Wl1, bl1r, Wr1, g1r, be1r)
    agg2 = _agg_call(src_p, dst_p, h1, zeros)
    z, pq = _tc_final_call(agg2, cntp, h1, Wl2, bl2r, Wr2,
                           ep_W1[:D, :], ep_W1[D:, :], ep_b1.reshape(1, D))
    pq_tab = pq.reshape(2 * N, D)

    es = jnp.concatenate([pos_edge_index[0], neg_edge_index[0],
                          jnp.zeros((EDPAD + 2 * CHD - ED,), i32)])
    ed = jnp.concatenate([pos_edge_index[1], neg_edge_index[1],
                          jnp.zeros((EDPAD + 2 * CHD - ED,), i32)])
    esd = jnp.stack([es, ed + N], axis=1).reshape(2 * (EDPAD + 2 * CHD))
    w2 = ep_W2[:, 0]
    b2v = jnp.broadcast_to(ep_b2, (16,)).astype(jnp.float32)
    scores = _decode_call(esd, pq_tab, w2, b2v)
    return z, scores[:100000], scores[100000:200000]


# speedup vs baseline: 1.0587x; 1.0587x over previous
import functools

import jax
import jax.numpy as jnp
from jax import lax
from jax.experimental import pallas as pl
from jax.experimental.pallas import tpu as pltpu
from jax.experimental.pallas import tpu_sc as plsc

N = 10000
D = 128
NPAD = 10112
DUMMY = 10016
ROWS_PT = NPAD // 16

E = 320000
CH = 96
NCH0 = 152
NCH1 = 60
EPAD = 16 * (NCH0 + NCH1) * CH

ED = 200000
CHD = 64
NCHD = 98
EDPT = NCHD * CHD
EDPAD = 32 * EDPT

_mesh = plsc.VectorSubcoreMesh(core_axis_name="c", subcore_axis_name="s")

_GDN = lax.GatherDimensionNumbers(
    offset_dims=(), collapsed_slice_dims=(0,), start_index_map=(0,))


def _permute(v, idx):
    return lax.gather(v, idx[:, None], _GDN, (1,),
                      mode=lax.GatherScatterMode.PROMISE_IN_BOUNDS)


def _agg_body(src_hbm, dst_hbm, x_hbm, zeros_hbm,
              agg_out,
              sidx_a, sidx_b, didx_a, didx_b, rows_a, rows_b, agg_sh,
              gsem_a, gsem_b, ssem_a, ssem_b, dsem_a, dsem_b):
    c = lax.axis_index("c")
    s = lax.axis_index("s")
    w = c * 16 + s
    rbase = s * ROWS_PT
    chunks = ((0, 96), (96, 96), (192, 96), (288, 96), (384, 96),
              (480, 96), (576, 56))
    for off, n in chunks:
        pltpu.sync_copy(zeros_hbm.at[pl.ds(rbase + off, n)],
                        rows_a.at[pl.ds(0, n)])
        pltpu.sync_copy(rows_a.at[pl.ds(0, n)],
                        agg_sh.at[pl.ds(rbase + off, n)])
    plsc.subcore_barrier()

    ebase = (s * (NCH0 + NCH1) + c * NCH0) * CH
    m = jnp.where(c == 0, NCH0, NCH1)
    sbufs = (sidx_a, sidx_b)
    dbufs = (didx_a, didx_b)
    rbufs = (rows_a, rows_b)
    gsems = (gsem_a, gsem_b)
    ssems = (ssem_a, ssem_b)
    dsems = (dsem_a, dsem_b)

    def sslice(i):
        return src_hbm.at[pl.ds(pl.multiple_of(ebase + i * CH, 8), CH)]

    def dslice(i):
        return dst_hbm.at[pl.ds(pl.multiple_of(ebase + i * CH, 8), CH)]

    def step(i, p):
        q = 1 - p
        pltpu.make_async_copy(sslice(i + 1), sbufs[q], ssems[q]).wait()
        pltpu.async_copy(x_hbm.at[sbufs[q]], rbufs[q], gsems[q])
        pltpu.make_async_copy(x_hbm.at[sbufs[p]], rbufs[p], gsems[p]).wait()
        pltpu.make_async_copy(dslice(i), dbufs[p], dsems[p]).wait()
        pltpu.sync_copy(rbufs[p], agg_sh.at[dbufs[p]], add=True)
        pltpu.async_copy(sslice(i + 2), sbufs[p], ssems[p])
        pltpu.async_copy(dslice(i + 2), dbufs[p], dsems[p])

    pltpu.sync_copy(sslice(0), sidx_a)
    pltpu.async_copy(dslice(0), didx_a, dsem_a)
    pltpu.async_copy(x_hbm.at[sidx_a], rows_a, gsem_a)
    pltpu.async_copy(sslice(1), sidx_b, ssem_b)
    pltpu.async_copy(dslice(1), didx_b, dsem_b)

    def body(g, carry):
        i = g * 2
        step(i, 0)
        step(i + 1, 1)
        return carry

    lax.fori_loop(0, m // 2 - 1, body, 0)
    step(m - 2, 0)
    pltpu.make_async_copy(x_hbm.at[sidx_b], rows_b, gsem_b).wait()
    pltpu.make_async_copy(dslice(m - 1), didx_b, dsem_b).wait()
    pltpu.sync_copy(rows_b, agg_sh.at[didx_b], add=True)
    pltpu.make_async_copy(sslice(m), sidx_a, ssem_a).wait()
    pltpu.make_async_copy(dslice(m), didx_a, dsem_a).wait()

    plsc.subcore_barrier()
    for off, n in chunks:
        pltpu.sync_copy(agg_sh.at[pl.ds(rbase + off, n)],
                        rows_a.at[pl.ds(0, n)])
        pltpu.sync_copy(rows_a.at[pl.ds(0, n)],
                        agg_out.at[c, pl.ds(rbase + off, n)])


_agg_call = pl.kernel(
    _agg_body,
    out_type=jax.ShapeDtypeStruct((2, NPAD, D), jnp.float32),
    mesh=_mesh,
    scratch_types=[
        pltpu.VMEM((CH,), jnp.int32),
        pltpu.VMEM((CH,), jnp.int32),
        pltpu.VMEM((CH,), jnp.int32),
        pltpu.VMEM((CH,), jnp.int32),
        pltpu.VMEM((CH, D), jnp.float32),
        pltpu.VMEM((CH, D), jnp.float32),
        pltpu.VMEM_SHARED((NPAD, D), jnp.float32),
        pltpu.SemaphoreType.DMA,
        pltpu.SemaphoreType.DMA,
        pltpu.SemaphoreType.DMA,
        pltpu.SemaphoreType.DMA,
        pltpu.SemaphoreType.DMA,
        pltpu.SemaphoreType.DMA,
    ],
)


def _decode_body(esd_hbm, pq_hbm, w2_hbm, b2_hbm,
                 out_hbm,
                 idx_a, idx_b, rows_a, rows_b, w2v, b2v, sc_v,
                 gsem_a, gsem_b, isem_a, isem_b):
    c = lax.axis_index("c")
    s = lax.axis_index("s")
    w = c * 16 + s
    ebase = w * EDPT
    pltpu.sync_copy(w2_hbm, w2v)
    pltpu.sync_copy(b2_hbm, b2v)
    w2r = [w2v[pl.ds(16 * k, 16)] for k in range(8)]
    b2r = b2v[...]
    lanes = lax.broadcasted_iota(jnp.int32, (16,), 0)
    ibufs = (idx_a, idx_b)
    isems = (isem_a, isem_b)
    rbufs = (rows_a, rows_b)
    gsems = (gsem_a, gsem_b)

    def isl(i):
        return esd_hbm.at[pl.ds(
            pl.multiple_of(2 * (ebase + i * CHD), 8), 2 * CHD)]

    def step(i, p):
        q = 1 - p
        pltpu.make_async_copy(isl(i + 1), ibufs[q], isems[q]).wait()
        pltpu.async_copy(pq_hbm.at[ibufs[q]], rbufs[q], gsems[q])
        pltpu.make_async_copy(pq_hbm.at[ibufs[p]], rbufs[p], gsems[p]).wait()
        pltpu.async_copy(isl(i + 2), ibufs[p], isems[p])
        rows = rbufs[p]

        def group(g, carry2):
            vec = jnp.zeros((16,), jnp.float32)
            for j in range(16):
                e = g * 16 + j
                acc = None
                for k in range(8):
                    pv = rows[2 * e, pl.ds(16 * k, 16)]
                    qv = rows[2 * e + 1, pl.ds(16 * k, 16)]
                    t = jnp.maximum(pv + qv, 0.0) * w2r[k]
                    acc = t if acc is None else acc + t
                for sh in (8, 4, 2, 1):
                    acc = acc + _permute(acc, lanes ^ sh)
                vec = jnp.where(lanes == j, acc, vec)
            vec = vec + b2r
            vec = 1.0 / (1.0 + jnp.exp(-vec))
            sc_v[pl.ds(g * 16, 16)] = vec
            return carry2

        lax.fori_loop(0, CHD // 16, group, 0)
        pltpu.sync_copy(sc_v, out_hbm.at[pl.ds(
            pl.multiple_of(ebase + i * CHD, 8), CHD)])

    pltpu.sync_copy(isl(0), idx_a)
    pltpu.async_copy(pq_hbm.at[idx_a], rows_a, gsem_a)
    pltpu.async_copy(isl(1), idx_b, isem_b)

    def chunk(g, carry):
        i = g * 2
        step(i, 0)
        step(i + 1, 1)
        return carry

    lax.fori_loop(0, NCHD // 2, chunk, 0)
    pltpu.make_async_copy(pq_hbm.at[idx_a], rows_a, gsem_a).wait()
    pltpu.make_async_copy(isl(NCHD + 1), idx_b, isem_b).wait()


_decode_call = pl.kernel(
    _decode_body,
    out_type=jax.ShapeDtypeStruct((EDPAD,), jnp.float32),
    mesh=_mesh,
    scratch_types=[
        pltpu.VMEM((2 * CHD,), jnp.int32),
        pltpu.VMEM((2 * CHD,), jnp.int32),
        pltpu.VMEM((2 * CHD, D), jnp.float32),
        pltpu.VMEM((2 * CHD, D), jnp.float32),
        pltpu.VMEM((D,), jnp.float32),
        pltpu.VMEM((16,), jnp.float32),
        pltpu.VMEM((CHD,), jnp.float32),
        pltpu.SemaphoreType.DMA,
        pltpu.SemaphoreType.DMA,
        pltpu.SemaphoreType.DMA,
        pltpu.SemaphoreType.DMA,
    ],
)


def _mean_from_parts(agg_ref, cnt_ref):
    a = agg_ref[0, :N, :] + agg_ref[1, :N, :]
    cnt = cnt_ref[0, :N, 0:1] + cnt_ref[1, :N, 0:1]
    return a / jnp.maximum(cnt, 1.0)


def _tc_layer_body(agg_ref, cnt_ref, h_ref, wl_ref, bl_ref, wr_ref,
                   g_ref, be_ref, out_ref):
    mean = _mean_from_parts(agg_ref, cnt_ref)
    h = h_ref[...]
    y = (jnp.dot(mean, wl_ref[...], preferred_element_type=jnp.float32)
         + bl_ref[...]
         + jnp.dot(h, wr_ref[...], preferred_element_type=jnp.float32))
    mu = jnp.mean(y, axis=0, keepdims=True)
    dlt = y - mu
    var = jnp.mean(dlt * dlt, axis=0, keepdims=True)
    yn = dlt * lax.rsqrt(var + 1e-5) * g_ref[...] + be_ref[...]
    out_ref[...] = jnp.maximum(yn, 0.0) + h


_tc_layer_call = pl.pallas_call(
    _tc_layer_body,
    out_shape=jax.ShapeDtypeStruct((N, D), jnp.float32),
)


def _tc_final_body(agg_ref, cnt_ref, h_ref, wl_ref, bl_ref, wr_ref,
                   w1a_ref, w1b_ref, b1_ref,
                   z_ref, pq_ref):
    mean = _mean_from_parts(agg_ref, cnt_ref)
    z = (jnp.dot(mean, wl_ref[...], preferred_element_type=jnp.float32)
         + bl_ref[...]
         + jnp.dot(h_ref[...], wr_ref[...], preferred_element_type=jnp.float32))
    z_ref[...] = z
    pq_ref[0] = jnp.dot(z, w1a_ref[...], preferred_element_type=jnp.float32)
    pq_ref[1] = (jnp.dot(z, w1b_ref[...], preferred_element_type=jnp.float32)
                 + b1_ref[...])


_tc_final_call = pl.pallas_call(
    _tc_final_body,
    out_shape=(jax.ShapeDtypeStruct((N, D), jnp.float32),
               jax.ShapeDtypeStruct((2, N, D), jnp.float32)),
)


def kernel(x, edge_index, pos_edge_index, neg_edge_index,
           Wl0, bl0, Wr0, Wl1, bl1, Wr1, Wl2, bl2, Wr2,
           g0, be0, g1, be1, ep_W1, ep_b1, ep_W2, ep_b2):
    i32 = jnp.int32
    src = edge_index[0].astype(i32)
    dst = edge_index[1].astype(i32)
    src_p = jnp.concatenate([src, jnp.zeros((EPAD + CH - E,), i32)])
    dst_p = jnp.concatenate([dst, jnp.full((EPAD - E,), DUMMY, i32),
                             jnp.zeros((CH,), i32)])
    zeros = jnp.zeros((NPAD, D), jnp.float32)
    ones_mat = jnp.ones((N, D), jnp.float32)

    bl0r, bl1r, bl2r = bl0.reshape(1, D), bl1.reshape(1, D), bl2.reshape(1, D)
    g0r, be0r = g0.reshape(1, D), be0.reshape(1, D)
    g1r, be1r = g1.reshape(1, D), be1.reshape(1, D)

    cntp = _agg_call(src_p, dst_p, ones_mat, zeros)
    agg0 = _agg_call(src_p, dst_p, x, zeros)
    h0 = _tc_layer_call(agg0, cntp, x, Wl0, bl0r, Wr0, g0r, be0r)
    agg1 = _agg_call(src_p, dst_p, h0, zeros)
    h1 = _tc_layer_call(agg1, cntp, h0, Wl1, bl1r, Wr1, g1r, be1r)
    agg2 = _agg_call(src_p, dst_p, h1, zeros)
    z, pq = _tc_final_call(agg2, cntp, h1, Wl2, bl2r, Wr2,
                           ep_W1[:D, :], ep_W1[D:, :], ep_b1.reshape(1, D))
    pq_tab = pq.reshape(2 * N, D)

    es = jnp.concatenate([pos_edge_index[0], neg_edge_index[0],
                          jnp.zeros((EDPAD + 2 * CHD - ED,), i32)])
    ed = jnp.concatenate([pos_edge_index[1], neg_edge_index[1],
                          jnp.zeros((EDPAD + 2 * CHD - ED,), i32)])
    esd = jnp.stack([es, ed + N], axis=1).reshape(2 * (EDPAD + 2 * CHD))
    w2 = ep_W2[:, 0]
    b2v = jnp.broadcast_to(ep_b2, (16,)).astype(jnp.float32)
    scores = _decode_call(esd, pq_tab, w2, b2v)
    return z, scores[:100000], scores[100000:200000]
